# baseline (speedup 1.0000x reference)
import functools

import jax
import jax.numpy as jnp
from jax import lax
from jax.experimental import pallas as pl
from jax.experimental.pallas import tpu as pltpu
from jax.experimental.pallas import tpu_sc as plsc

_B = 16384
_D = 128
_V = 100000
_NW = 32
_BPW = _B // _NW
_C = 64
_NCHUNK = _BPW // _C
_NBUF = 4
_IC = 128
_NIDX = _BPW // _IC
_LANES = 16


def _sc_body(inp, inp2, uemb, pemb, bias,
             partials, bsum_out,
             idx_u, idx_p, idx_pb,
             u_buf0, u_buf1, u_buf2, u_buf3,
             p_buf0, p_buf1, p_buf2, p_buf3,
             ub_v, pb_v, bsum_v, acc_v,
             sem_u0, sem_u1, sem_u2, sem_u3,
             sem_p0, sem_p1, sem_p2, sem_p3, sem_b):
    cid = lax.axis_index("c")
    sid = lax.axis_index("s")
    wid = sid * 2 + cid
    base = wid * _BPW

    stage_u = pltpu.async_copy(inp.at[wid], idx_u, sem_b)
    stage_p = pltpu.async_copy(inp2.at[wid], idx_p, sem_b)
    stage_u.wait()
    stage_p.wait()

    def idx_slice(ref, c):
        blk, off = (c * _C) // _IC, (c * _C) % _IC
        return ref.at[blk, pl.ds(off, _C)]

    u_bufs = (u_buf0, u_buf1, u_buf2, u_buf3)
    p_bufs = (p_buf0, p_buf1, p_buf2, p_buf3)
    sems_u = (sem_u0, sem_u1, sem_u2, sem_u3)
    sems_p = (sem_p0, sem_p1, sem_p2, sem_p3)

    def fire(c):
        s = c % _NBUF
        return (
            pltpu.async_copy(uemb.at[idx_slice(idx_u, c)], u_bufs[s], sems_u[s]),
            pltpu.async_copy(pemb.at[idx_slice(idx_p, c)], p_bufs[s], sems_p[s]),
        )

    cps = {}
    for c in range(_NBUF - 1):
        cps[c] = fire(c)

    for blk in range(_NIDX):
        for g in range(_IC // _LANES):
            sl = pl.ds(g * _LANES, _LANES)
            idx_pb[blk, sl] = idx_p[blk, sl] + _V

    bias_cps = []
    for c in range(_NIDX):
        sl = pl.ds(c * _IC, _IC)
        bias_cps.append(pltpu.async_copy(
            bias.at[idx_u.at[c]], ub_v.at[sl], sem_b))
        bias_cps.append(pltpu.async_copy(
            bias.at[idx_pb.at[c]], pb_v.at[sl], sem_b))

    accs = tuple(jnp.zeros((_LANES,), jnp.float32) for _ in range(_D // _LANES))
    for c in range(_NCHUNK):
        s = c % _NBUF
        if c + _NBUF - 1 < _NCHUNK:
            cps[c + _NBUF - 1] = fire(c + _NBUF - 1)
        cps[c][0].wait()
        cps[c][1].wait()

        ue_ref = u_bufs[s]
        pe_ref = p_bufs[s]

        def row_body(r, a, ue_ref=ue_ref, pe_ref=pe_ref):
            new = []
            for j in range(_D // _LANES):
                uv = ue_ref[r, pl.ds(j * _LANES, _LANES)]
                pv = pe_ref[r, pl.ds(j * _LANES, _LANES)]
                new.append(a[j] + uv * pv)
            return tuple(new)

        accs = lax.fori_loop(0, _C, row_body, accs)

    total = accs[0]
    for j in range(1, _D // _LANES):
        total = total + accs[j]
    acc_v[...] = total

    for cp in bias_cps:
        cp.wait()
    for g in range(_BPW // _LANES):
        sl = pl.ds(g * _LANES, _LANES)
        bsum_v[sl] = ub_v[sl] + pb_v[sl]

    pltpu.sync_copy(acc_v, partials.at[pl.ds(wid * _LANES, _LANES)])
    pltpu.sync_copy(bsum_v, bsum_out.at[pl.ds(base, _BPW)])


@functools.partial(
    pl.kernel,
    mesh=plsc.VectorSubcoreMesh(core_axis_name="c", subcore_axis_name="s"),
    out_type=[
        jax.ShapeDtypeStruct((_NW * _LANES,), jnp.float32),
        jax.ShapeDtypeStruct((_B,), jnp.float32),
    ],
    scratch_types=[
        pltpu.VMEM((_NIDX, _IC), jnp.int32),
        pltpu.VMEM((_NIDX, _IC), jnp.int32),
        pltpu.VMEM((_NIDX, _IC), jnp.int32),
        pltpu.VMEM((_C, _D), jnp.float32),
        pltpu.VMEM((_C, _D), jnp.float32),
        pltpu.VMEM((_C, _D), jnp.float32),
        pltpu.VMEM((_C, _D), jnp.float32),
        pltpu.VMEM((_C, _D), jnp.float32),
        pltpu.VMEM((_C, _D), jnp.float32),
        pltpu.VMEM((_C, _D), jnp.float32),
        pltpu.VMEM((_C, _D), jnp.float32),
        pltpu.VMEM((_BPW,), jnp.float32),
        pltpu.VMEM((_BPW,), jnp.float32),
        pltpu.VMEM((_BPW,), jnp.float32),
        pltpu.VMEM((_LANES,), jnp.float32),
        pltpu.SemaphoreType.DMA,
        pltpu.SemaphoreType.DMA,
        pltpu.SemaphoreType.DMA,
        pltpu.SemaphoreType.DMA,
        pltpu.SemaphoreType.DMA,
        pltpu.SemaphoreType.DMA,
        pltpu.SemaphoreType.DMA,
        pltpu.SemaphoreType.DMA,
        pltpu.SemaphoreType.DMA,
    ],
)
def _sc_gather_dot(inp, inp2, uemb, pemb, bias, *rest):
    _sc_body(inp, inp2, uemb, pemb, bias, *rest)


def _tc_finish(partials_ref, bsum_ref, out_ref):
    s = jnp.sum(partials_ref[...])
    out_ref[...] = jax.nn.sigmoid(bsum_ref[...] + s)


def kernel(inputs, user_embedding, user_bias, product_embedding, product_bias):
    u_idx = inputs[:, 0].astype(jnp.int32).reshape(_NW, _NIDX, _IC)
    p_idx = inputs[:, 1].astype(jnp.int32).reshape(_NW, _NIDX, _IC)
    bias_flat = jnp.concatenate([user_bias, product_bias]).reshape(-1)
    partials, bsum = _sc_gather_dot(
        u_idx, p_idx, user_embedding, product_embedding, bias_flat)

    out = pl.pallas_call(
        _tc_finish,
        out_shape=jax.ShapeDtypeStruct((_B // _D, _D), jnp.float32),
    )(partials.reshape(4, _D), bsum.reshape(_B // _D, _D))
    return out.reshape(_B, 1)


# speedup vs baseline: 1.1376x; 1.1376x over previous
import functools

import jax
import jax.numpy as jnp
from jax import lax
from jax.experimental import pallas as pl
from jax.experimental.pallas import tpu as pltpu
from jax.experimental.pallas import tpu_sc as plsc

_B = 16384
_D = 128
_V = 100000
_NW = 32
_BPW = _B // _NW
_C = 64
_NCHUNK = _BPW // _C
_NBUF = 4
_IC = 128
_NIDX = _BPW // _IC
_LANES = 16


def _sc_body(inp, inp2, uemb, pemb, ubias, pbias,
             partials, bsum_out,
             idx_u, idx_p,
             u_buf0, u_buf1, u_buf2, u_buf3,
             p_buf0, p_buf1, p_buf2, p_buf3,
             ub_v, pb_v, bsum_v, acc_v,
             sem_u0, sem_u1, sem_u2, sem_u3,
             sem_p0, sem_p1, sem_p2, sem_p3, sem_b):
    cid = lax.axis_index("c")
    sid = lax.axis_index("s")
    wid = sid * 2 + cid
    base = wid * _BPW

    stage_u = pltpu.async_copy(inp.at[wid], idx_u, sem_b)
    stage_p = pltpu.async_copy(inp2.at[wid], idx_p, sem_b)
    stage_u.wait()
    stage_p.wait()

    def idx_slice(ref, c):
        blk, off = (c * _C) // _IC, (c * _C) % _IC
        return ref.at[blk, pl.ds(off, _C)]

    u_bufs = (u_buf0, u_buf1, u_buf2, u_buf3)
    p_bufs = (p_buf0, p_buf1, p_buf2, p_buf3)
    sems_u = (sem_u0, sem_u1, sem_u2, sem_u3)
    sems_p = (sem_p0, sem_p1, sem_p2, sem_p3)

    def fire(c):
        s = c % _NBUF
        return (
            pltpu.async_copy(uemb.at[idx_slice(idx_u, c)], u_bufs[s], sems_u[s]),
            pltpu.async_copy(pemb.at[idx_slice(idx_p, c)], p_bufs[s], sems_p[s]),
        )

    cps = {}
    for c in range(_NBUF - 1):
        cps[c] = fire(c)

    bias_cps = []
    for c in range(_NIDX):
        sl = pl.ds(c * _IC, _IC)
        bias_cps.append(pltpu.async_copy(
            ubias.at[idx_u.at[c]], ub_v.at[sl], sem_b))
        bias_cps.append(pltpu.async_copy(
            pbias.at[idx_p.at[c]], pb_v.at[sl], sem_b))

    accs = tuple(jnp.zeros((_LANES,), jnp.float32) for _ in range(_D // _LANES))
    for c in range(_NCHUNK):
        s = c % _NBUF
        if c + _NBUF - 1 < _NCHUNK:
            cps[c + _NBUF - 1] = fire(c + _NBUF - 1)
        cps[c][0].wait()
        cps[c][1].wait()

        ue_ref = u_bufs[s]
        pe_ref = p_bufs[s]

        def row_body(r, a, ue_ref=ue_ref, pe_ref=pe_ref):
            new = []
            for j in range(_D // _LANES):
                uv = ue_ref[r, pl.ds(j * _LANES, _LANES)]
                pv = pe_ref[r, pl.ds(j * _LANES, _LANES)]
                new.append(a[j] + uv * pv)
            return tuple(new)

        accs = lax.fori_loop(0, _C, row_body, accs)

    total = accs[0]
    for j in range(1, _D // _LANES):
        total = total + accs[j]
    acc_v[...] = total

    for cp in bias_cps:
        cp.wait()
    for g in range(_BPW // _LANES):
        sl = pl.ds(g * _LANES, _LANES)
        bsum_v[sl] = ub_v[sl] + pb_v[sl]

    pltpu.sync_copy(acc_v, partials.at[pl.ds(wid * _LANES, _LANES)])
    pltpu.sync_copy(bsum_v, bsum_out.at[pl.ds(base, _BPW)])


@functools.partial(
    pl.kernel,
    mesh=plsc.VectorSubcoreMesh(core_axis_name="c", subcore_axis_name="s"),
    out_type=[
        jax.ShapeDtypeStruct((_NW * _LANES,), jnp.float32),
        jax.ShapeDtypeStruct((_B,), jnp.float32),
    ],
    scratch_types=[
        pltpu.VMEM((_NIDX, _IC), jnp.int32),
        pltpu.VMEM((_NIDX, _IC), jnp.int32),
        pltpu.VMEM((_C, _D), jnp.float32),
        pltpu.VMEM((_C, _D), jnp.float32),
        pltpu.VMEM((_C, _D), jnp.float32),
        pltpu.VMEM((_C, _D), jnp.float32),
        pltpu.VMEM((_C, _D), jnp.float32),
        pltpu.VMEM((_C, _D), jnp.float32),
        pltpu.VMEM((_C, _D), jnp.float32),
        pltpu.VMEM((_C, _D), jnp.float32),
        pltpu.VMEM((_BPW,), jnp.float32),
        pltpu.VMEM((_BPW,), jnp.float32),
        pltpu.VMEM((_BPW,), jnp.float32),
        pltpu.VMEM((_LANES,), jnp.float32),
        pltpu.SemaphoreType.DMA,
        pltpu.SemaphoreType.DMA,
        pltpu.SemaphoreType.DMA,
        pltpu.SemaphoreType.DMA,
        pltpu.SemaphoreType.DMA,
        pltpu.SemaphoreType.DMA,
        pltpu.SemaphoreType.DMA,
        pltpu.SemaphoreType.DMA,
        pltpu.SemaphoreType.DMA,
    ],
)
def _sc_gather_dot(inp, inp2, uemb, pemb, ubias, pbias, *rest):
    _sc_body(inp, inp2, uemb, pemb, ubias, pbias, *rest)


def _tc_finish(partials_ref, bsum_ref, out_ref):
    s = jnp.sum(partials_ref[...])
    out_ref[...] = jax.nn.sigmoid(bsum_ref[...] + s)


def kernel(inputs, user_embedding, user_bias, product_embedding, product_bias):
    u_idx = inputs[:, 0].astype(jnp.int32).reshape(_NW, _NIDX, _IC)
    p_idx = inputs[:, 1].astype(jnp.int32).reshape(_NW, _NIDX, _IC)
    partials, bsum = _sc_gather_dot(
        u_idx, p_idx, user_embedding, product_embedding,
        user_bias.reshape(-1), product_bias.reshape(-1))

    out = pl.pallas_call(
        _tc_finish,
        out_shape=jax.ShapeDtypeStruct((_B // _D, _D), jnp.float32),
    )(partials.reshape(4, _D), bsum.reshape(_B // _D, _D))
    return out.reshape(_B, 1)
